# Initial kernel scaffold; baseline (speedup 1.0000x reference)
import jax
import jax.numpy as jnp
from jax.experimental import pallas as pl


def kernel(x, mask, params):
    raise NotImplementedError("write your pallas kernel here")


# speedup vs baseline: 1.1461x; 1.1461x over previous
import functools

import jax
import jax.numpy as jnp
from jax.experimental import pallas as pl
from jax.experimental.pallas import tpu as pltpu

B, T, F = 8, 256, 32
H, L = 128, 32
KNN = 5
PREC = jax.lax.Precision.HIGHEST


def _knn_body(x_ref, m_ref, o_ref, *, k):
    xp = x_ref[0]
    mp = m_ref[0]
    n = xp.shape[0]
    u = xp * xp * mp
    v = xp * mp
    cnt = jnp.dot(mp, mp.T, precision=PREC)
    sq = (jnp.dot(u, mp.T, precision=PREC)
          + jnp.dot(mp, u.T, precision=PREC)
          - 2.0 * jnp.dot(v, v.T, precision=PREC))
    ratio = sq / (cnt + 1e-8)
    pos = ratio > 0.0
    safe = jnp.where(pos, ratio, 1.0)
    d = jnp.where(cnt > 0.0, jnp.where(pos, jnp.sqrt(safe), 0.0), jnp.inf)
    col = jax.lax.broadcasted_iota(jnp.int32, (n, n), 1)
    row = jax.lax.broadcasted_iota(jnp.int32, (n, n), 0)
    d = jnp.where(row == col, jnp.inf, d)

    cur = d
    idxs, ds = [], []
    for _ in range(k):
        mval = jnp.min(cur, axis=-1, keepdims=True)
        idx = jnp.min(jnp.where(cur == mval, col, n), -1, keepdims=True)
        idxs.append(idx)
        ds.append(mval)
        cur = jnp.where(col == idx, jnp.inf, cur)
    top_d = jnp.concatenate(ds, axis=-1)
    valid = top_d < jnp.inf
    safe_d = jnp.where(valid, top_d, 0.0)
    vc = jnp.sum(valid.astype(jnp.float32), -1, keepdims=True)
    sigma = jnp.sum(safe_d, -1, keepdims=True) / (vc + 1e-8)
    sigma = jnp.where(vc > 0.0, sigma, 1.0)
    w = jnp.exp(-jnp.square(safe_d) / (jnp.square(sigma) + 1e-8))
    w = jnp.where(valid, w, 0.0)
    w = w / (jnp.sum(w, -1, keepdims=True) + 1e-8)

    wmat = jnp.zeros((n, n), jnp.float32)
    for j in range(k):
        wmat = wmat + jnp.where(col == idxs[j], w[:, j:j + 1], 0.0)
    nv = jnp.dot(wmat, v, precision=PREC)
    ws = jnp.dot(wmat, mp, precision=PREC)
    filled = nv / (ws + 1e-8)
    o_ref[0] = jnp.where(mp > 0.0, xp, jnp.where(ws > 0.0, filled, 0.0))


def _knn_fill(xp, mp, k):
    b, n, dd = xp.shape
    spec = pl.BlockSpec((1, n, dd), lambda i: (i, 0, 0))
    return pl.pallas_call(
        functools.partial(_knn_body, k=k),
        grid=(b,),
        in_specs=[spec, spec],
        out_specs=spec,
        out_shape=jax.ShapeDtypeStruct((b, n, dd), jnp.float32),
    )(xp, mp)


def _bd_w(ws, units):
    e_n = len(ws)
    din = ws[0].shape[0]
    out = jnp.zeros((e_n * din, 4 * e_n * units), jnp.float32)
    for e, w in enumerate(ws):
        for g in range(4):
            out = out.at[e * din:(e + 1) * din,
                         g * e_n * units + e * units:
                         g * e_n * units + (e + 1) * units].set(
                             w[:, g * units:(g + 1) * units])
    return out


def _bd_b(bs, units):
    e_n = len(bs)
    out = jnp.zeros((1, 4 * e_n * units), jnp.float32)
    for e, b in enumerate(bs):
        for g in range(4):
            out = out.at[0, g * e_n * units + e * units:
                         g * e_n * units + (e + 1) * units].set(
                             b[g * units:(g + 1) * units])
    return out


def _cat_rows(vs):
    return jnp.concatenate(vs, axis=0)[None, :]


def _ln_blocks(hseq, g_row, b_row, e_n, units):
    outs = []
    for e in range(e_n):
        sl = hseq[:, :, e * units:(e + 1) * units]
        mu = jnp.mean(sl, -1, keepdims=True)
        var = jnp.mean(jnp.square(sl - mu), -1, keepdims=True)
        outs.append((sl - mu) / jnp.sqrt(var + 1e-3))
    hn = jnp.concatenate(outs, axis=-1) if e_n > 1 else outs[0]
    return hn * g_row + b_row


def _lstm_scan(xz_ref, rk, hbuf_ref, nb, eh):
    def step(t, carry):
        h, c = carry
        z = xz_ref[pl.ds(t, 1)][0] + jnp.dot(h, rk, precision=PREC)
        i = jax.nn.sigmoid(z[:, 0 * eh:1 * eh])
        f = jax.nn.sigmoid(z[:, 1 * eh:2 * eh])
        g = jnp.tanh(z[:, 2 * eh:3 * eh])
        o = jax.nn.sigmoid(z[:, 3 * eh:4 * eh])
        c2 = f * c + i * g
        h2 = o * jnp.tanh(c2)
        hbuf_ref[pl.ds(t, 1)] = h2[None]
        return (h2, c2)
    zero = jnp.zeros((nb, eh), jnp.float32)
    jax.lax.fori_loop(0, T, step, (zero, zero))


def _enc_body(xin_ref, k1_ref, b1_ref, rk1_ref, k2_ref, b2_ref, rk2_ref,
              g1_ref, be1_ref, g2_ref, be2_ref, lw_ref, lb_ref,
              o_ref, xz_ref, hbuf_ref):
    e_n, eh = 3, 3 * H
    din = xin_ref.shape[-1]
    xin = xin_ref[...]
    xz = jnp.dot(xin.reshape(T * B, din), k1_ref[...], precision=PREC)
    xz_ref[...] = (xz + b1_ref[...]).reshape(T, B, 4 * eh)
    _lstm_scan(xz_ref, rk1_ref[...], hbuf_ref, B, eh)

    h1 = _ln_blocks(hbuf_ref[...], g1_ref[...], be1_ref[...], e_n, H)
    xz2 = jnp.dot(h1.reshape(T * B, eh), k2_ref[...], precision=PREC)
    xz_ref[...] = (xz2 + b2_ref[...]).reshape(T, B, 4 * eh)
    _lstm_scan(xz_ref, rk2_ref[...], hbuf_ref, B, eh)

    h2 = _ln_blocks(hbuf_ref[...], g2_ref[...], be2_ref[...], e_n, H)
    z = jnp.dot(h2.reshape(T * B, eh), lw_ref[...], precision=PREC)
    o_ref[...] = (z + lb_ref[...]).reshape(T, B, e_n * L)


def _run_encoders(xin, enc_ps):
    eh = 3 * H
    k1 = _bd_w([p['l1_k'] for p in enc_ps], H)
    b1 = _bd_b([p['l1_b'] for p in enc_ps], H)
    rk1 = _bd_w([p['l1_rk'] for p in enc_ps], H)
    k2 = _bd_w([p['l2_k'] for p in enc_ps], H)
    b2 = _bd_b([p['l2_b'] for p in enc_ps], H)
    rk2 = _bd_w([p['l2_rk'] for p in enc_ps], H)
    g1 = _cat_rows([p['n1_g'] for p in enc_ps])
    be1 = _cat_rows([p['n1_b'] for p in enc_ps])
    g2 = _cat_rows([p['n2_g'] for p in enc_ps])
    be2 = _cat_rows([p['n2_b'] for p in enc_ps])
    lw = jax.scipy.linalg.block_diag(*[p['lat_w'] for p in enc_ps])
    lb = _cat_rows([p['lat_b'] for p in enc_ps])
    return pl.pallas_call(
        _enc_body,
        out_shape=jax.ShapeDtypeStruct((T, B, 3 * L), jnp.float32),
        scratch_shapes=[pltpu.VMEM((T, B, 4 * eh), jnp.float32),
                        pltpu.VMEM((T, B, eh), jnp.float32)],
    )(xin, k1, b1, rk1, k2, b2, rk2, g1, be1, g2, be2, lw, lb)


def _dec_body(zall_ref, mask_ref, d1w_ref, d1b_ref, d2w_ref, d2b_ref,
              aw_ref, ab_ref, k1_ref, b1_ref, rk1_ref, k2_ref, b2_ref,
              rk2_ref, g1_ref, be1_ref, g2_ref, be2_ref, ow_ref, ob_ref,
              o_ref, xz_ref, hbuf_ref):
    zall = zall_ref[...]
    pooled = jnp.mean(zall, axis=0)
    mr = 1.0 - jnp.mean(jnp.mean(mask_ref[...], axis=2), axis=1,
                        keepdims=True)
    ginp = jnp.concatenate([pooled, mr], axis=-1)
    h = jnp.dot(ginp, d1w_ref[...], precision=PREC) + d1b_ref[...]
    h = 1.0 - jnp.exp(-jnp.square(h))
    h = jnp.dot(h, d2w_ref[...], precision=PREC) + d2b_ref[...]
    h = 1.0 - jnp.exp(-jnp.square(h))
    lg = jnp.dot(h, aw_ref[...], precision=PREC) + ab_ref[...]
    lg = lg - jnp.max(lg, -1, keepdims=True)
    ex = jnp.exp(lg)
    alpha = ex / jnp.sum(ex, -1, keepdims=True)

    zf = (zall[:, :, 0 * L:1 * L] * alpha[:, 0:1]
          + zall[:, :, 1 * L:2 * L] * alpha[:, 1:2]
          + zall[:, :, 2 * L:3 * L] * alpha[:, 2:3])

    xz = jnp.dot(zf.reshape(T * B, L), k1_ref[...], precision=PREC)
    xz_ref[...] = (xz + b1_ref[...]).reshape(T, B, 4 * H)
    _lstm_scan(xz_ref, rk1_ref[...], hbuf_ref, B, H)

    h1 = _ln_blocks(hbuf_ref[...], g1_ref[...], be1_ref[...], 1, H)
    xz2 = jnp.dot(h1.reshape(T * B, H), k2_ref[...], precision=PREC)
    xz_ref[...] = (xz2 + b2_ref[...]).reshape(T, B, 4 * H)
    _lstm_scan(xz_ref, rk2_ref[...], hbuf_ref, B, H)

    h2 = _ln_blocks(hbuf_ref[...], g2_ref[...], be2_ref[...], 1, H)
    out = jnp.dot(h2.reshape(T * B, H), ow_ref[...], precision=PREC)
    out = jnp.clip(out + ob_ref[...], -5.0, 5.0)
    o_ref[...] = out.reshape(T, B, F)


def _run_decoder(zall, mask, gp, dp):
    return pl.pallas_call(
        _dec_body,
        out_shape=jax.ShapeDtypeStruct((T, B, F), jnp.float32),
        scratch_shapes=[pltpu.VMEM((T, B, 4 * H), jnp.float32),
                        pltpu.VMEM((T, B, H), jnp.float32)],
    )(zall, mask,
      gp['d1_w'], gp['d1_b'][None, :], gp['d2_w'], gp['d2_b'][None, :],
      gp['a_w'], gp['a_b'][None, :],
      dp['l1_k'], dp['l1_b'][None, :], dp['l1_rk'],
      dp['l2_k'], dp['l2_b'][None, :], dp['l2_rk'],
      dp['n1_g'][None, :], dp['n1_b'][None, :],
      dp['n2_g'][None, :], dp['n2_b'][None, :],
      dp['out_w'], dp['out_b'][None, :])


def kernel(x, mask, params):
    x_space = _knn_fill(x, mask, KNN)
    xt = _knn_fill(jnp.transpose(x, (0, 2, 1)),
                   jnp.transpose(mask, (0, 2, 1)), KNN)
    x_time = jnp.transpose(xt, (0, 2, 1))
    x_zero = x * mask
    xin = jnp.concatenate([x_zero, mask, x_space, mask, x_time, mask], -1)
    xin = jnp.transpose(xin, (1, 0, 2))
    zall = _run_encoders(
        xin, [params['enc_orig'], params['enc_space'], params['enc_time']])
    out = _run_decoder(zall, mask, params['gate'], params['dec'])
    return jnp.transpose(out, (1, 0, 2))


# speedup vs baseline: 2.5233x; 2.2016x over previous
import functools

import jax
import jax.numpy as jnp
from jax.experimental import pallas as pl
from jax.experimental.pallas import tpu as pltpu

B, T, F = 8, 256, 32
H, L = 128, 32
KNN = 5
PREC = jax.lax.Precision.HIGHEST
PREC_NN = jax.lax.Precision.DEFAULT


def _knn_body(x_ref, m_ref, o_ref, *, k):
    xp = x_ref[0]
    mp = m_ref[0]
    n = xp.shape[0]
    u = xp * xp * mp
    v = xp * mp
    cnt = jnp.dot(mp, mp.T, precision=PREC)
    sq = (jnp.dot(u, mp.T, precision=PREC)
          + jnp.dot(mp, u.T, precision=PREC)
          - 2.0 * jnp.dot(v, v.T, precision=PREC))
    ratio = sq / (cnt + 1e-8)
    pos = ratio > 0.0
    safe = jnp.where(pos, ratio, 1.0)
    d = jnp.where(cnt > 0.0, jnp.where(pos, jnp.sqrt(safe), 0.0), jnp.inf)
    col = jax.lax.broadcasted_iota(jnp.int32, (n, n), 1)
    row = jax.lax.broadcasted_iota(jnp.int32, (n, n), 0)
    d = jnp.where(row == col, jnp.inf, d)

    cur = d
    idxs, ds = [], []
    for _ in range(k):
        mval = jnp.min(cur, axis=-1, keepdims=True)
        idx = jnp.min(jnp.where(cur == mval, col, n), -1, keepdims=True)
        idxs.append(idx)
        ds.append(mval)
        cur = jnp.where(col == idx, jnp.inf, cur)
    top_d = jnp.concatenate(ds, axis=-1)
    valid = top_d < jnp.inf
    safe_d = jnp.where(valid, top_d, 0.0)
    vc = jnp.sum(valid.astype(jnp.float32), -1, keepdims=True)
    sigma = jnp.sum(safe_d, -1, keepdims=True) / (vc + 1e-8)
    sigma = jnp.where(vc > 0.0, sigma, 1.0)
    w = jnp.exp(-jnp.square(safe_d) / (jnp.square(sigma) + 1e-8))
    w = jnp.where(valid, w, 0.0)
    w = w / (jnp.sum(w, -1, keepdims=True) + 1e-8)

    wmat = jnp.zeros((n, n), jnp.float32)
    for j in range(k):
        wmat = wmat + jnp.where(col == idxs[j], w[:, j:j + 1], 0.0)
    nv = jnp.dot(wmat, v, precision=PREC)
    ws = jnp.dot(wmat, mp, precision=PREC)
    filled = nv / (ws + 1e-8)
    o_ref[0] = jnp.where(mp > 0.0, xp, jnp.where(ws > 0.0, filled, 0.0))


def _knn_fill(xp, mp, k):
    b, n, dd = xp.shape
    spec = pl.BlockSpec((1, n, dd), lambda i: (i, 0, 0))
    return pl.pallas_call(
        functools.partial(_knn_body, k=k),
        grid=(b,),
        in_specs=[spec, spec],
        out_specs=spec,
        out_shape=jax.ShapeDtypeStruct((b, n, dd), jnp.float32),
    )(xp, mp)


def _bd_w(ws, units):
    e_n = len(ws)
    din = ws[0].shape[0]
    out = jnp.zeros((e_n * din, 4 * e_n * units), jnp.float32)
    for e, w in enumerate(ws):
        for g in range(4):
            out = out.at[e * din:(e + 1) * din,
                         g * e_n * units + e * units:
                         g * e_n * units + (e + 1) * units].set(
                             w[:, g * units:(g + 1) * units])
    return out


def _bd_b(bs, units):
    e_n = len(bs)
    out = jnp.zeros((1, 4 * e_n * units), jnp.float32)
    for e, b in enumerate(bs):
        for g in range(4):
            out = out.at[0, g * e_n * units + e * units:
                         g * e_n * units + (e + 1) * units].set(
                             b[g * units:(g + 1) * units])
    return out


def _cat_rows(vs):
    return jnp.concatenate(vs, axis=0)[None, :]


def _ln_blocks(hseq, g_row, b_row, e_n, units):
    outs = []
    for e in range(e_n):
        sl = hseq[:, :, e * units:(e + 1) * units]
        mu = jnp.mean(sl, -1, keepdims=True)
        var = jnp.mean(jnp.square(sl - mu), -1, keepdims=True)
        outs.append((sl - mu) / jnp.sqrt(var + 1e-3))
    hn = jnp.concatenate(outs, axis=-1) if e_n > 1 else outs[0]
    return hn * g_row + b_row


def _lstm_scan(xz_ref, rk, hbuf_ref, nb, eh):
    def step(t, carry):
        h, c = carry
        z = xz_ref[pl.ds(t, 1)][0] + jnp.dot(h, rk, precision=PREC_NN)
        i = jax.nn.sigmoid(z[:, 0 * eh:1 * eh])
        f = jax.nn.sigmoid(z[:, 1 * eh:2 * eh])
        g = jnp.tanh(z[:, 2 * eh:3 * eh])
        o = jax.nn.sigmoid(z[:, 3 * eh:4 * eh])
        c2 = f * c + i * g
        h2 = o * jnp.tanh(c2)
        hbuf_ref[pl.ds(t, 1)] = h2[None]
        return (h2, c2)
    zero = jnp.zeros((nb, eh), jnp.float32)
    jax.lax.fori_loop(0, T, step, (zero, zero))


def _enc_body(xin_ref, k1_ref, b1_ref, rk1_ref, k2_ref, b2_ref, rk2_ref,
              g1_ref, be1_ref, g2_ref, be2_ref, lw_ref, lb_ref,
              o_ref, xz_ref, hbuf_ref):
    e_n, eh = 3, 3 * H
    din = xin_ref.shape[-1]
    xin = xin_ref[...]
    xz = jnp.dot(xin.reshape(T * B, din), k1_ref[...], precision=PREC_NN)
    xz_ref[...] = (xz + b1_ref[...]).reshape(T, B, 4 * eh)
    _lstm_scan(xz_ref, rk1_ref[...], hbuf_ref, B, eh)

    h1 = _ln_blocks(hbuf_ref[...], g1_ref[...], be1_ref[...], e_n, H)
    xz2 = jnp.dot(h1.reshape(T * B, eh), k2_ref[...], precision=PREC_NN)
    xz_ref[...] = (xz2 + b2_ref[...]).reshape(T, B, 4 * eh)
    _lstm_scan(xz_ref, rk2_ref[...], hbuf_ref, B, eh)

    h2 = _ln_blocks(hbuf_ref[...], g2_ref[...], be2_ref[...], e_n, H)
    z = jnp.dot(h2.reshape(T * B, eh), lw_ref[...], precision=PREC_NN)
    o_ref[...] = (z + lb_ref[...]).reshape(T, B, e_n * L)


def _run_encoders(xin, enc_ps):
    eh = 3 * H
    k1 = _bd_w([p['l1_k'] for p in enc_ps], H)
    b1 = _bd_b([p['l1_b'] for p in enc_ps], H)
    rk1 = _bd_w([p['l1_rk'] for p in enc_ps], H)
    k2 = _bd_w([p['l2_k'] for p in enc_ps], H)
    b2 = _bd_b([p['l2_b'] for p in enc_ps], H)
    rk2 = _bd_w([p['l2_rk'] for p in enc_ps], H)
    g1 = _cat_rows([p['n1_g'] for p in enc_ps])
    be1 = _cat_rows([p['n1_b'] for p in enc_ps])
    g2 = _cat_rows([p['n2_g'] for p in enc_ps])
    be2 = _cat_rows([p['n2_b'] for p in enc_ps])
    lw = jax.scipy.linalg.block_diag(*[p['lat_w'] for p in enc_ps])
    lb = _cat_rows([p['lat_b'] for p in enc_ps])
    return pl.pallas_call(
        _enc_body,
        out_shape=jax.ShapeDtypeStruct((T, B, 3 * L), jnp.float32),
        scratch_shapes=[pltpu.VMEM((T, B, 4 * eh), jnp.float32),
                        pltpu.VMEM((T, B, eh), jnp.float32)],
    )(xin, k1, b1, rk1, k2, b2, rk2, g1, be1, g2, be2, lw, lb)


def _dec_body(zall_ref, mask_ref, d1w_ref, d1b_ref, d2w_ref, d2b_ref,
              aw_ref, ab_ref, k1_ref, b1_ref, rk1_ref, k2_ref, b2_ref,
              rk2_ref, g1_ref, be1_ref, g2_ref, be2_ref, ow_ref, ob_ref,
              o_ref, xz_ref, hbuf_ref):
    zall = zall_ref[...]
    pooled = jnp.mean(zall, axis=0)
    mr = 1.0 - jnp.mean(jnp.mean(mask_ref[...], axis=2), axis=1,
                        keepdims=True)
    ginp = jnp.concatenate([pooled, mr], axis=-1)
    h = jnp.dot(ginp, d1w_ref[...], precision=PREC_NN) + d1b_ref[...]
    h = 1.0 - jnp.exp(-jnp.square(h))
    h = jnp.dot(h, d2w_ref[...], precision=PREC_NN) + d2b_ref[...]
    h = 1.0 - jnp.exp(-jnp.square(h))
    lg = jnp.dot(h, aw_ref[...], precision=PREC_NN) + ab_ref[...]
    lg = lg - jnp.max(lg, -1, keepdims=True)
    ex = jnp.exp(lg)
    alpha = ex / jnp.sum(ex, -1, keepdims=True)

    zf = (zall[:, :, 0 * L:1 * L] * alpha[:, 0:1]
          + zall[:, :, 1 * L:2 * L] * alpha[:, 1:2]
          + zall[:, :, 2 * L:3 * L] * alpha[:, 2:3])

    xz = jnp.dot(zf.reshape(T * B, L), k1_ref[...], precision=PREC_NN)
    xz_ref[...] = (xz + b1_ref[...]).reshape(T, B, 4 * H)
    _lstm_scan(xz_ref, rk1_ref[...], hbuf_ref, B, H)

    h1 = _ln_blocks(hbuf_ref[...], g1_ref[...], be1_ref[...], 1, H)
    xz2 = jnp.dot(h1.reshape(T * B, H), k2_ref[...], precision=PREC_NN)
    xz_ref[...] = (xz2 + b2_ref[...]).reshape(T, B, 4 * H)
    _lstm_scan(xz_ref, rk2_ref[...], hbuf_ref, B, H)

    h2 = _ln_blocks(hbuf_ref[...], g2_ref[...], be2_ref[...], 1, H)
    out = jnp.dot(h2.reshape(T * B, H), ow_ref[...], precision=PREC_NN)
    out = jnp.clip(out + ob_ref[...], -5.0, 5.0)
    o_ref[...] = out.reshape(T, B, F)


def _run_decoder(zall, mask, gp, dp):
    return pl.pallas_call(
        _dec_body,
        out_shape=jax.ShapeDtypeStruct((T, B, F), jnp.float32),
        scratch_shapes=[pltpu.VMEM((T, B, 4 * H), jnp.float32),
                        pltpu.VMEM((T, B, H), jnp.float32)],
    )(zall, mask,
      gp['d1_w'], gp['d1_b'][None, :], gp['d2_w'], gp['d2_b'][None, :],
      gp['a_w'], gp['a_b'][None, :],
      dp['l1_k'], dp['l1_b'][None, :], dp['l1_rk'],
      dp['l2_k'], dp['l2_b'][None, :], dp['l2_rk'],
      dp['n1_g'][None, :], dp['n1_b'][None, :],
      dp['n2_g'][None, :], dp['n2_b'][None, :],
      dp['out_w'], dp['out_b'][None, :])


def kernel(x, mask, params):
    x_space = _knn_fill(x, mask, KNN)
    xt = _knn_fill(jnp.transpose(x, (0, 2, 1)),
                   jnp.transpose(mask, (0, 2, 1)), KNN)
    x_time = jnp.transpose(xt, (0, 2, 1))
    x_zero = x * mask
    xin = jnp.concatenate([x_zero, mask, x_space, mask, x_time, mask], -1)
    xin = jnp.transpose(xin, (1, 0, 2))
    zall = _run_encoders(
        xin, [params['enc_orig'], params['enc_space'], params['enc_time']])
    out = _run_decoder(zall, mask, params['gate'], params['dec'])
    return jnp.transpose(out, (1, 0, 2))


# speedup vs baseline: 4.2365x; 1.6790x over previous
import functools

import jax
import jax.numpy as jnp
from jax.experimental import pallas as pl
from jax.experimental.pallas import tpu as pltpu

B, T, F = 8, 256, 32
H, L = 128, 32
KNN = 5
PREC = jax.lax.Precision.HIGHEST
PREC_NN = jax.lax.Precision.DEFAULT


def _knn_body(x_ref, m_ref, o_ref, *, k):
    xp = x_ref[0]
    mp = m_ref[0]
    n = xp.shape[0]
    u = xp * xp * mp
    v = xp * mp
    cnt = jnp.dot(mp, mp.T, precision=PREC)
    sq = (jnp.dot(u, mp.T, precision=PREC)
          + jnp.dot(mp, u.T, precision=PREC)
          - 2.0 * jnp.dot(v, v.T, precision=PREC))
    ratio = sq / (cnt + 1e-8)
    pos = ratio > 0.0
    safe = jnp.where(pos, ratio, 1.0)
    d = jnp.where(cnt > 0.0, jnp.where(pos, jnp.sqrt(safe), 0.0), jnp.inf)
    col = jax.lax.broadcasted_iota(jnp.int32, (n, n), 1)
    row = jax.lax.broadcasted_iota(jnp.int32, (n, n), 0)
    d = jnp.where(row == col, jnp.inf, d)

    cur = d
    idxs, ds = [], []
    for _ in range(k):
        mval = jnp.min(cur, axis=-1, keepdims=True)
        idx = jnp.min(jnp.where(cur == mval, col, n), -1, keepdims=True)
        idxs.append(idx)
        ds.append(mval)
        cur = jnp.where(col == idx, jnp.inf, cur)
    top_d = jnp.concatenate(ds, axis=-1)
    valid = top_d < jnp.inf
    safe_d = jnp.where(valid, top_d, 0.0)
    vc = jnp.sum(valid.astype(jnp.float32), -1, keepdims=True)
    sigma = jnp.sum(safe_d, -1, keepdims=True) / (vc + 1e-8)
    sigma = jnp.where(vc > 0.0, sigma, 1.0)
    w = jnp.exp(-jnp.square(safe_d) / (jnp.square(sigma) + 1e-8))
    w = jnp.where(valid, w, 0.0)
    w = w / (jnp.sum(w, -1, keepdims=True) + 1e-8)

    wmat = jnp.zeros((n, n), jnp.float32)
    for j in range(k):
        wmat = wmat + jnp.where(col == idxs[j], w[:, j:j + 1], 0.0)
    nv = jnp.dot(wmat, v, precision=PREC)
    ws = jnp.dot(wmat, mp, precision=PREC)
    filled = nv / (ws + 1e-8)
    o_ref[0] = jnp.where(mp > 0.0, xp, jnp.where(ws > 0.0, filled, 0.0))


def _knn_fill(xp, mp, k):
    b, n, dd = xp.shape
    spec = pl.BlockSpec((1, n, dd), lambda i: (i, 0, 0))
    return pl.pallas_call(
        functools.partial(_knn_body, k=k),
        grid=(b,),
        in_specs=[spec, spec],
        out_specs=spec,
        out_shape=jax.ShapeDtypeStruct((b, n, dd), jnp.float32),
    )(xp, mp)


def _cat_rows(vs):
    return jnp.concatenate(vs, axis=0)[None, :]


def _ln_blocks(hseq, g_row, b_row, e_n, units):
    outs = []
    for e in range(e_n):
        sl = hseq[:, :, e * units:(e + 1) * units]
        mu = jnp.mean(sl, -1, keepdims=True)
        var = jnp.mean(jnp.square(sl - mu), -1, keepdims=True)
        outs.append((sl - mu) / jnp.sqrt(var + 1e-3))
    hn = jnp.concatenate(outs, axis=-1) if e_n > 1 else outs[0]
    return hn * g_row + b_row


def _lstm_scan(xz_ref, rks, hbuf_ref, nb, units):
    e_n = len(rks)
    def step(t, carry):
        hs, cs = carry
        xz = xz_ref[pl.ds(t, 1)][0]
        new_h, new_c = [], []
        for e in range(e_n):
            z = (xz[:, e * 4 * units:(e + 1) * 4 * units]
                 + jnp.dot(hs[e], rks[e], precision=PREC_NN))
            i = jax.nn.sigmoid(z[:, 0 * units:1 * units])
            f = jax.nn.sigmoid(z[:, 1 * units:2 * units])
            g = jnp.tanh(z[:, 2 * units:3 * units])
            o = jax.nn.sigmoid(z[:, 3 * units:4 * units])
            c2 = f * cs[e] + i * g
            new_c.append(c2)
            new_h.append(o * jnp.tanh(c2))
        hcat = new_h[0] if e_n == 1 else jnp.concatenate(new_h, -1)
        hbuf_ref[pl.ds(t, 1)] = hcat[None]
        return (tuple(new_h), tuple(new_c))
    zero = jnp.zeros((nb, units), jnp.float32)
    jax.lax.fori_loop(0, T, step,
                      ((zero,) * e_n, (zero,) * e_n))


def _enc_body(xin_ref, k1_ref, b1_ref, rk1a_ref, rk1b_ref, rk1c_ref,
              k2_ref, b2_ref, rk2a_ref, rk2b_ref, rk2c_ref,
              g1_ref, be1_ref, g2_ref, be2_ref, lw_ref, lb_ref,
              o_ref, xz_ref, hbuf_ref):
    e_n, eh = 3, 3 * H
    din = xin_ref.shape[-1]
    xin = xin_ref[...]
    xz = jnp.dot(xin.reshape(T * B, din), k1_ref[...], precision=PREC_NN)
    xz_ref[...] = (xz + b1_ref[...]).reshape(T, B, 4 * eh)
    _lstm_scan(xz_ref, [rk1a_ref[...], rk1b_ref[...], rk1c_ref[...]],
               hbuf_ref, B, H)

    h1 = _ln_blocks(hbuf_ref[...], g1_ref[...], be1_ref[...], e_n, H)
    xz2 = jnp.dot(h1.reshape(T * B, eh), k2_ref[...], precision=PREC_NN)
    xz_ref[...] = (xz2 + b2_ref[...]).reshape(T, B, 4 * eh)
    _lstm_scan(xz_ref, [rk2a_ref[...], rk2b_ref[...], rk2c_ref[...]],
               hbuf_ref, B, H)

    h2 = _ln_blocks(hbuf_ref[...], g2_ref[...], be2_ref[...], e_n, H)
    z = jnp.dot(h2.reshape(T * B, eh), lw_ref[...], precision=PREC_NN)
    o_ref[...] = (z + lb_ref[...]).reshape(T, B, e_n * L)


def _run_encoders(xin, enc_ps):
    eh = 3 * H
    k1 = jax.scipy.linalg.block_diag(*[p['l1_k'] for p in enc_ps])
    b1 = _cat_rows([p['l1_b'] for p in enc_ps])
    k2 = jax.scipy.linalg.block_diag(*[p['l2_k'] for p in enc_ps])
    b2 = _cat_rows([p['l2_b'] for p in enc_ps])
    g1 = _cat_rows([p['n1_g'] for p in enc_ps])
    be1 = _cat_rows([p['n1_b'] for p in enc_ps])
    g2 = _cat_rows([p['n2_g'] for p in enc_ps])
    be2 = _cat_rows([p['n2_b'] for p in enc_ps])
    lw = jax.scipy.linalg.block_diag(*[p['lat_w'] for p in enc_ps])
    lb = _cat_rows([p['lat_b'] for p in enc_ps])
    return pl.pallas_call(
        _enc_body,
        out_shape=jax.ShapeDtypeStruct((T, B, 3 * L), jnp.float32),
        scratch_shapes=[pltpu.VMEM((T, B, 4 * eh), jnp.float32),
                        pltpu.VMEM((T, B, eh), jnp.float32)],
    )(xin, k1, b1,
      enc_ps[0]['l1_rk'], enc_ps[1]['l1_rk'], enc_ps[2]['l1_rk'],
      k2, b2,
      enc_ps[0]['l2_rk'], enc_ps[1]['l2_rk'], enc_ps[2]['l2_rk'],
      g1, be1, g2, be2, lw, lb)


def _dec_body(zall_ref, mask_ref, d1w_ref, d1b_ref, d2w_ref, d2b_ref,
              aw_ref, ab_ref, k1_ref, b1_ref, rk1_ref, k2_ref, b2_ref,
              rk2_ref, g1_ref, be1_ref, g2_ref, be2_ref, ow_ref, ob_ref,
              o_ref, xz_ref, hbuf_ref):
    zall = zall_ref[...]
    pooled = jnp.mean(zall, axis=0)
    mr = 1.0 - jnp.mean(jnp.mean(mask_ref[...], axis=2), axis=1,
                        keepdims=True)
    ginp = jnp.concatenate([pooled, mr], axis=-1)
    h = jnp.dot(ginp, d1w_ref[...], precision=PREC_NN) + d1b_ref[...]
    h = 1.0 - jnp.exp(-jnp.square(h))
    h = jnp.dot(h, d2w_ref[...], precision=PREC_NN) + d2b_ref[...]
    h = 1.0 - jnp.exp(-jnp.square(h))
    lg = jnp.dot(h, aw_ref[...], precision=PREC_NN) + ab_ref[...]
    lg = lg - jnp.max(lg, -1, keepdims=True)
    ex = jnp.exp(lg)
    alpha = ex / jnp.sum(ex, -1, keepdims=True)

    zf = (zall[:, :, 0 * L:1 * L] * alpha[:, 0:1]
          + zall[:, :, 1 * L:2 * L] * alpha[:, 1:2]
          + zall[:, :, 2 * L:3 * L] * alpha[:, 2:3])

    xz = jnp.dot(zf.reshape(T * B, L), k1_ref[...], precision=PREC_NN)
    xz_ref[...] = (xz + b1_ref[...]).reshape(T, B, 4 * H)
    _lstm_scan(xz_ref, [rk1_ref[...]], hbuf_ref, B, H)

    h1 = _ln_blocks(hbuf_ref[...], g1_ref[...], be1_ref[...], 1, H)
    xz2 = jnp.dot(h1.reshape(T * B, H), k2_ref[...], precision=PREC_NN)
    xz_ref[...] = (xz2 + b2_ref[...]).reshape(T, B, 4 * H)
    _lstm_scan(xz_ref, [rk2_ref[...]], hbuf_ref, B, H)

    h2 = _ln_blocks(hbuf_ref[...], g2_ref[...], be2_ref[...], 1, H)
    out = jnp.dot(h2.reshape(T * B, H), ow_ref[...], precision=PREC_NN)
    out = jnp.clip(out + ob_ref[...], -5.0, 5.0)
    o_ref[...] = out.reshape(T, B, F)


def _run_decoder(zall, mask, gp, dp):
    return pl.pallas_call(
        _dec_body,
        out_shape=jax.ShapeDtypeStruct((T, B, F), jnp.float32),
        scratch_shapes=[pltpu.VMEM((T, B, 4 * H), jnp.float32),
                        pltpu.VMEM((T, B, H), jnp.float32)],
    )(zall, mask,
      gp['d1_w'], gp['d1_b'][None, :], gp['d2_w'], gp['d2_b'][None, :],
      gp['a_w'], gp['a_b'][None, :],
      dp['l1_k'], dp['l1_b'][None, :], dp['l1_rk'],
      dp['l2_k'], dp['l2_b'][None, :], dp['l2_rk'],
      dp['n1_g'][None, :], dp['n1_b'][None, :],
      dp['n2_g'][None, :], dp['n2_b'][None, :],
      dp['out_w'], dp['out_b'][None, :])


def kernel(x, mask, params):
    x_space = _knn_fill(x, mask, KNN)
    xt = _knn_fill(jnp.transpose(x, (0, 2, 1)),
                   jnp.transpose(mask, (0, 2, 1)), KNN)
    x_time = jnp.transpose(xt, (0, 2, 1))
    x_zero = x * mask
    xin = jnp.concatenate([x_zero, mask, x_space, mask, x_time, mask], -1)
    xin = jnp.transpose(xin, (1, 0, 2))
    zall = _run_encoders(
        xin, [params['enc_orig'], params['enc_space'], params['enc_time']])
    out = _run_decoder(zall, mask, params['gate'], params['dec'])
    return jnp.transpose(out, (1, 0, 2))


# speedup vs baseline: 4.5391x; 1.0714x over previous
import functools

import jax
import jax.numpy as jnp
from jax.experimental import pallas as pl
from jax.experimental.pallas import tpu as pltpu

B, T, F = 8, 256, 32
H, L = 128, 32
KNN = 5
PREC = jax.lax.Precision.HIGHEST
PREC_NN = jax.lax.Precision.DEFAULT


def _knn_body(x_ref, m_ref, o_ref, *, k):
    xp = x_ref[0]
    mp = m_ref[0]
    n = xp.shape[0]
    u = xp * xp * mp
    v = xp * mp
    cnt = jnp.dot(mp, mp.T, precision=PREC)
    sq = (jnp.dot(u, mp.T, precision=PREC)
          + jnp.dot(mp, u.T, precision=PREC)
          - 2.0 * jnp.dot(v, v.T, precision=PREC))
    ratio = sq / (cnt + 1e-8)
    pos = ratio > 0.0
    safe = jnp.where(pos, ratio, 1.0)
    d = jnp.where(cnt > 0.0, jnp.where(pos, jnp.sqrt(safe), 0.0), jnp.inf)
    col = jax.lax.broadcasted_iota(jnp.int32, (n, n), 1)
    row = jax.lax.broadcasted_iota(jnp.int32, (n, n), 0)
    d = jnp.where(row == col, jnp.inf, d)

    cur = d
    idxs, ds = [], []
    for _ in range(k):
        mval = jnp.min(cur, axis=-1, keepdims=True)
        idx = jnp.min(jnp.where(cur == mval, col, n), -1, keepdims=True)
        idxs.append(idx)
        ds.append(mval)
        cur = jnp.where(col == idx, jnp.inf, cur)
    top_d = jnp.concatenate(ds, axis=-1)
    valid = top_d < jnp.inf
    safe_d = jnp.where(valid, top_d, 0.0)
    vc = jnp.sum(valid.astype(jnp.float32), -1, keepdims=True)
    sigma = jnp.sum(safe_d, -1, keepdims=True) / (vc + 1e-8)
    sigma = jnp.where(vc > 0.0, sigma, 1.0)
    w = jnp.exp(-jnp.square(safe_d) / (jnp.square(sigma) + 1e-8))
    w = jnp.where(valid, w, 0.0)
    w = w / (jnp.sum(w, -1, keepdims=True) + 1e-8)

    wmat = jnp.zeros((n, n), jnp.float32)
    for j in range(k):
        wmat = wmat + jnp.where(col == idxs[j], w[:, j:j + 1], 0.0)
    nv = jnp.dot(wmat, v, precision=PREC)
    ws = jnp.dot(wmat, mp, precision=PREC)
    filled = nv / (ws + 1e-8)
    o_ref[0] = jnp.where(mp > 0.0, xp, jnp.where(ws > 0.0, filled, 0.0))


def _knn_fill(xp, mp, k):
    b, n, dd = xp.shape
    spec = pl.BlockSpec((1, n, dd), lambda i: (i, 0, 0))
    return pl.pallas_call(
        functools.partial(_knn_body, k=k),
        grid=(b,),
        in_specs=[spec, spec],
        out_specs=spec,
        out_shape=jax.ShapeDtypeStruct((b, n, dd), jnp.float32),
    )(xp, mp)


def _cat_rows(vs):
    return jnp.concatenate(vs, axis=0)[None, :]


def _ln_blocks(hseq, g_row, b_row, e_n, units):
    outs = []
    for e in range(e_n):
        sl = hseq[:, :, e * units:(e + 1) * units]
        mu = jnp.mean(sl, -1, keepdims=True)
        var = jnp.mean(jnp.square(sl - mu), -1, keepdims=True)
        outs.append((sl - mu) / jnp.sqrt(var + 1e-3))
    hn = jnp.concatenate(outs, axis=-1) if e_n > 1 else outs[0]
    return hn * g_row + b_row


def _lstm_scan(xz_ref, rks, hbuf_ref, nb, units):
    e_n = len(rks)
    halves = [(rk[:, :2 * units].astype(jnp.bfloat16),
               rk[:, 2 * units:].astype(jnp.bfloat16)) for rk in rks]
    def step(t, carry):
        hs, cs = carry
        xz = xz_ref[pl.ds(t, 1)][0]
        new_h, new_c = [], []
        for e in range(e_n):
            hb = hs[e].astype(jnp.bfloat16)
            lo, hi = halves[e]
            base = e * 4 * units
            zlo = (xz[:, base:base + 2 * units]
                   + jnp.dot(hb, lo, preferred_element_type=jnp.float32))
            zhi = (xz[:, base + 2 * units:base + 4 * units]
                   + jnp.dot(hb, hi, preferred_element_type=jnp.float32))
            i = jax.nn.sigmoid(zlo[:, :units])
            f = jax.nn.sigmoid(zlo[:, units:])
            g = jnp.tanh(zhi[:, :units])
            o = jax.nn.sigmoid(zhi[:, units:])
            c2 = f * cs[e] + i * g
            new_c.append(c2)
            new_h.append(o * jnp.tanh(c2))
        hcat = new_h[0] if e_n == 1 else jnp.concatenate(new_h, -1)
        hbuf_ref[pl.ds(t, 1)] = hcat[None]
        return (tuple(new_h), tuple(new_c))
    zero = jnp.zeros((nb, units), jnp.float32)
    jax.lax.fori_loop(0, T, step,
                      ((zero,) * e_n, (zero,) * e_n), unroll=2)


def _enc_body(xin_ref, k1_ref, b1_ref, rk1a_ref, rk1b_ref, rk1c_ref,
              k2_ref, b2_ref, rk2a_ref, rk2b_ref, rk2c_ref,
              g1_ref, be1_ref, g2_ref, be2_ref, lw_ref, lb_ref,
              o_ref, xz_ref, hbuf_ref):
    e_n, eh = 3, 3 * H
    din = xin_ref.shape[-1]
    xin = xin_ref[...]
    xz = jnp.dot(xin.reshape(T * B, din), k1_ref[...], precision=PREC_NN)
    xz_ref[...] = (xz + b1_ref[...]).reshape(T, B, 4 * eh)
    _lstm_scan(xz_ref, [rk1a_ref[...], rk1b_ref[...], rk1c_ref[...]],
               hbuf_ref, B, H)

    h1 = _ln_blocks(hbuf_ref[...], g1_ref[...], be1_ref[...], e_n, H)
    xz2 = jnp.dot(h1.reshape(T * B, eh), k2_ref[...], precision=PREC_NN)
    xz_ref[...] = (xz2 + b2_ref[...]).reshape(T, B, 4 * eh)
    _lstm_scan(xz_ref, [rk2a_ref[...], rk2b_ref[...], rk2c_ref[...]],
               hbuf_ref, B, H)

    h2 = _ln_blocks(hbuf_ref[...], g2_ref[...], be2_ref[...], e_n, H)
    z = jnp.dot(h2.reshape(T * B, eh), lw_ref[...], precision=PREC_NN)
    o_ref[...] = (z + lb_ref[...]).reshape(T, B, e_n * L)


def _run_encoders(xin, enc_ps):
    eh = 3 * H
    k1 = jax.scipy.linalg.block_diag(*[p['l1_k'] for p in enc_ps])
    b1 = _cat_rows([p['l1_b'] for p in enc_ps])
    k2 = jax.scipy.linalg.block_diag(*[p['l2_k'] for p in enc_ps])
    b2 = _cat_rows([p['l2_b'] for p in enc_ps])
    g1 = _cat_rows([p['n1_g'] for p in enc_ps])
    be1 = _cat_rows([p['n1_b'] for p in enc_ps])
    g2 = _cat_rows([p['n2_g'] for p in enc_ps])
    be2 = _cat_rows([p['n2_b'] for p in enc_ps])
    lw = jax.scipy.linalg.block_diag(*[p['lat_w'] for p in enc_ps])
    lb = _cat_rows([p['lat_b'] for p in enc_ps])
    return pl.pallas_call(
        _enc_body,
        out_shape=jax.ShapeDtypeStruct((T, B, 3 * L), jnp.float32),
        scratch_shapes=[pltpu.VMEM((T, B, 4 * eh), jnp.float32),
                        pltpu.VMEM((T, B, eh), jnp.float32)],
    )(xin, k1, b1,
      enc_ps[0]['l1_rk'], enc_ps[1]['l1_rk'], enc_ps[2]['l1_rk'],
      k2, b2,
      enc_ps[0]['l2_rk'], enc_ps[1]['l2_rk'], enc_ps[2]['l2_rk'],
      g1, be1, g2, be2, lw, lb)


def _dec_body(zall_ref, mask_ref, d1w_ref, d1b_ref, d2w_ref, d2b_ref,
              aw_ref, ab_ref, k1_ref, b1_ref, rk1_ref, k2_ref, b2_ref,
              rk2_ref, g1_ref, be1_ref, g2_ref, be2_ref, ow_ref, ob_ref,
              o_ref, xz_ref, hbuf_ref):
    zall = zall_ref[...]
    pooled = jnp.mean(zall, axis=0)
    mr = 1.0 - jnp.mean(jnp.mean(mask_ref[...], axis=2), axis=1,
                        keepdims=True)
    ginp = jnp.concatenate([pooled, mr], axis=-1)
    h = jnp.dot(ginp, d1w_ref[...], precision=PREC_NN) + d1b_ref[...]
    h = 1.0 - jnp.exp(-jnp.square(h))
    h = jnp.dot(h, d2w_ref[...], precision=PREC_NN) + d2b_ref[...]
    h = 1.0 - jnp.exp(-jnp.square(h))
    lg = jnp.dot(h, aw_ref[...], precision=PREC_NN) + ab_ref[...]
    lg = lg - jnp.max(lg, -1, keepdims=True)
    ex = jnp.exp(lg)
    alpha = ex / jnp.sum(ex, -1, keepdims=True)

    zf = (zall[:, :, 0 * L:1 * L] * alpha[:, 0:1]
          + zall[:, :, 1 * L:2 * L] * alpha[:, 1:2]
          + zall[:, :, 2 * L:3 * L] * alpha[:, 2:3])

    xz = jnp.dot(zf.reshape(T * B, L), k1_ref[...], precision=PREC_NN)
    xz_ref[...] = (xz + b1_ref[...]).reshape(T, B, 4 * H)
    _lstm_scan(xz_ref, [rk1_ref[...]], hbuf_ref, B, H)

    h1 = _ln_blocks(hbuf_ref[...], g1_ref[...], be1_ref[...], 1, H)
    xz2 = jnp.dot(h1.reshape(T * B, H), k2_ref[...], precision=PREC_NN)
    xz_ref[...] = (xz2 + b2_ref[...]).reshape(T, B, 4 * H)
    _lstm_scan(xz_ref, [rk2_ref[...]], hbuf_ref, B, H)

    h2 = _ln_blocks(hbuf_ref[...], g2_ref[...], be2_ref[...], 1, H)
    out = jnp.dot(h2.reshape(T * B, H), ow_ref[...], precision=PREC_NN)
    out = jnp.clip(out + ob_ref[...], -5.0, 5.0)
    o_ref[...] = out.reshape(T, B, F)


def _run_decoder(zall, mask, gp, dp):
    return pl.pallas_call(
        _dec_body,
        out_shape=jax.ShapeDtypeStruct((T, B, F), jnp.float32),
        scratch_shapes=[pltpu.VMEM((T, B, 4 * H), jnp.float32),
                        pltpu.VMEM((T, B, H), jnp.float32)],
    )(zall, mask,
      gp['d1_w'], gp['d1_b'][None, :], gp['d2_w'], gp['d2_b'][None, :],
      gp['a_w'], gp['a_b'][None, :],
      dp['l1_k'], dp['l1_b'][None, :], dp['l1_rk'],
      dp['l2_k'], dp['l2_b'][None, :], dp['l2_rk'],
      dp['n1_g'][None, :], dp['n1_b'][None, :],
      dp['n2_g'][None, :], dp['n2_b'][None, :],
      dp['out_w'], dp['out_b'][None, :])


def kernel(x, mask, params):
    x_space = _knn_fill(x, mask, KNN)
    xt = _knn_fill(jnp.transpose(x, (0, 2, 1)),
                   jnp.transpose(mask, (0, 2, 1)), KNN)
    x_time = jnp.transpose(xt, (0, 2, 1))
    x_zero = x * mask
    xin = jnp.concatenate([x_zero, mask, x_space, mask, x_time, mask], -1)
    xin = jnp.transpose(xin, (1, 0, 2))
    zall = _run_encoders(
        xin, [params['enc_orig'], params['enc_space'], params['enc_time']])
    out = _run_decoder(zall, mask, params['gate'], params['dec'])
    return jnp.transpose(out, (1, 0, 2))


# speedup vs baseline: 4.6736x; 1.0296x over previous
import functools

import jax
import jax.numpy as jnp
from jax.experimental import pallas as pl
from jax.experimental.pallas import tpu as pltpu

B, T, F = 8, 256, 32
H, L = 128, 32
KNN = 5
PREC = jax.lax.Precision.HIGHEST
PREC_NN = jax.lax.Precision.DEFAULT


def _knn_rows(xp, mp, k):
    n = xp.shape[0]
    u = xp * xp * mp
    v = xp * mp
    cnt = jnp.dot(mp, mp.T, precision=PREC)
    sq = (jnp.dot(u, mp.T, precision=PREC)
          + jnp.dot(mp, u.T, precision=PREC)
          - 2.0 * jnp.dot(v, v.T, precision=PREC))
    ratio = sq / (cnt + 1e-8)
    pos = ratio > 0.0
    safe = jnp.where(pos, ratio, 1.0)
    d = jnp.where(cnt > 0.0, jnp.where(pos, jnp.sqrt(safe), 0.0), jnp.inf)
    col = jax.lax.broadcasted_iota(jnp.int32, (n, n), 1)
    row = jax.lax.broadcasted_iota(jnp.int32, (n, n), 0)
    d = jnp.where(row == col, jnp.inf, d)

    cur = d
    idxs, ds = [], []
    for _ in range(k):
        mval = jnp.min(cur, axis=-1, keepdims=True)
        idx = jnp.min(jnp.where(cur == mval, col, n), -1, keepdims=True)
        idxs.append(idx)
        ds.append(mval)
        cur = jnp.where(col == idx, jnp.inf, cur)
    top_d = jnp.concatenate(ds, axis=-1)
    valid = top_d < jnp.inf
    safe_d = jnp.where(valid, top_d, 0.0)
    vc = jnp.sum(valid.astype(jnp.float32), -1, keepdims=True)
    sigma = jnp.sum(safe_d, -1, keepdims=True) / (vc + 1e-8)
    sigma = jnp.where(vc > 0.0, sigma, 1.0)
    w = jnp.exp(-jnp.square(safe_d) / (jnp.square(sigma) + 1e-8))
    w = jnp.where(valid, w, 0.0)
    w = w / (jnp.sum(w, -1, keepdims=True) + 1e-8)

    wmat = jnp.zeros((n, n), jnp.float32)
    for j in range(k):
        wmat = wmat + jnp.where(col == idxs[j], w[:, j:j + 1], 0.0)
    nv = jnp.dot(wmat, v, precision=PREC)
    ws = jnp.dot(wmat, mp, precision=PREC)
    filled = nv / (ws + 1e-8)
    return jnp.where(mp > 0.0, xp, jnp.where(ws > 0.0, filled, 0.0))


def _cat_rows(vs):
    return jnp.concatenate(vs, axis=0)[None, :]


def _ln_blocks(hseq, g_row, b_row, e_n, units):
    outs = []
    for e in range(e_n):
        sl = hseq[:, :, e * units:(e + 1) * units]
        mu = jnp.mean(sl, -1, keepdims=True)
        var = jnp.mean(jnp.square(sl - mu), -1, keepdims=True)
        outs.append((sl - mu) / jnp.sqrt(var + 1e-3))
    hn = jnp.concatenate(outs, axis=-1) if e_n > 1 else outs[0]
    return hn * g_row + b_row


def _lstm_scan(xz_ref, rks, hbuf_ref, nb, units):
    e_n = len(rks)
    halves = [(rk[:, :2 * units].astype(jnp.bfloat16),
               rk[:, 2 * units:].astype(jnp.bfloat16)) for rk in rks]
    def step(t, carry):
        hs, cs = carry
        xz = xz_ref[pl.ds(t, 1)][0]
        new_h, new_c = [], []
        for e in range(e_n):
            hb = hs[e].astype(jnp.bfloat16)
            lo, hi = halves[e]
            base = e * 4 * units
            zlo = (xz[:, base:base + 2 * units]
                   + jnp.dot(hb, lo, preferred_element_type=jnp.float32))
            zhi = (xz[:, base + 2 * units:base + 4 * units]
                   + jnp.dot(hb, hi, preferred_element_type=jnp.float32))
            i = jax.nn.sigmoid(zlo[:, :units])
            f = jax.nn.sigmoid(zlo[:, units:])
            g = jnp.tanh(zhi[:, :units])
            o = jax.nn.sigmoid(zhi[:, units:])
            c2 = f * cs[e] + i * g
            new_c.append(c2)
            new_h.append(o * jnp.tanh(c2))
        hcat = new_h[0] if e_n == 1 else jnp.concatenate(new_h, -1)
        hbuf_ref[pl.ds(t, 1)] = hcat[None]
        return (tuple(new_h), tuple(new_c))
    zero = jnp.zeros((nb, units), jnp.float32)
    jax.lax.fori_loop(0, T, step,
                      ((zero,) * e_n, (zero,) * e_n), unroll=2)


def _enc_body(x_ref, m_ref, k1_ref, b1_ref, rk1a_ref, rk1b_ref, rk1c_ref,
              k2_ref, b2_ref, rk2a_ref, rk2b_ref, rk2c_ref,
              g1_ref, be1_ref, g2_ref, be2_ref, lw_ref, lb_ref,
              o_ref, xz_ref, hbuf_ref, xin_ref):
    e_n, eh = 3, 3 * H
    for b in range(B):
        xb = x_ref[b]
        mb = m_ref[b]
        xsp = _knn_rows(xb, mb, KNN)
        xtm = jnp.transpose(
            _knn_rows(jnp.transpose(xb), jnp.transpose(mb), KNN))
        xin_ref[:, b:b + 1, 0 * F:1 * F] = (xb * mb)[:, None, :]
        xin_ref[:, b:b + 1, 1 * F:2 * F] = mb[:, None, :]
        xin_ref[:, b:b + 1, 2 * F:3 * F] = xsp[:, None, :]
        xin_ref[:, b:b + 1, 3 * F:4 * F] = mb[:, None, :]
        xin_ref[:, b:b + 1, 4 * F:5 * F] = xtm[:, None, :]
        xin_ref[:, b:b + 1, 5 * F:6 * F] = mb[:, None, :]
    din = 6 * F
    xin = xin_ref[...]
    xz = jnp.dot(xin.reshape(T * B, din), k1_ref[...], precision=PREC_NN)
    xz_ref[...] = (xz + b1_ref[...]).reshape(T, B, 4 * eh)
    _lstm_scan(xz_ref, [rk1a_ref[...], rk1b_ref[...], rk1c_ref[...]],
               hbuf_ref, B, H)

    h1 = _ln_blocks(hbuf_ref[...], g1_ref[...], be1_ref[...], e_n, H)
    xz2 = jnp.dot(h1.reshape(T * B, eh), k2_ref[...], precision=PREC_NN)
    xz_ref[...] = (xz2 + b2_ref[...]).reshape(T, B, 4 * eh)
    _lstm_scan(xz_ref, [rk2a_ref[...], rk2b_ref[...], rk2c_ref[...]],
               hbuf_ref, B, H)

    h2 = _ln_blocks(hbuf_ref[...], g2_ref[...], be2_ref[...], e_n, H)
    z = jnp.dot(h2.reshape(T * B, eh), lw_ref[...], precision=PREC_NN)
    o_ref[...] = (z + lb_ref[...]).reshape(T, B, e_n * L)


def _run_encoders(x, mask, enc_ps):
    eh = 3 * H
    k1 = jax.scipy.linalg.block_diag(*[p['l1_k'] for p in enc_ps])
    b1 = _cat_rows([p['l1_b'] for p in enc_ps])
    k2 = jax.scipy.linalg.block_diag(*[p['l2_k'] for p in enc_ps])
    b2 = _cat_rows([p['l2_b'] for p in enc_ps])
    g1 = _cat_rows([p['n1_g'] for p in enc_ps])
    be1 = _cat_rows([p['n1_b'] for p in enc_ps])
    g2 = _cat_rows([p['n2_g'] for p in enc_ps])
    be2 = _cat_rows([p['n2_b'] for p in enc_ps])
    lw = jax.scipy.linalg.block_diag(*[p['lat_w'] for p in enc_ps])
    lb = _cat_rows([p['lat_b'] for p in enc_ps])
    return pl.pallas_call(
        _enc_body,
        out_shape=jax.ShapeDtypeStruct((T, B, 3 * L), jnp.float32),
        scratch_shapes=[pltpu.VMEM((T, B, 4 * eh), jnp.float32),
                        pltpu.VMEM((T, B, eh), jnp.float32),
                        pltpu.VMEM((T, B, 6 * F), jnp.float32)],
    )(x, mask, k1, b1,
      enc_ps[0]['l1_rk'], enc_ps[1]['l1_rk'], enc_ps[2]['l1_rk'],
      k2, b2,
      enc_ps[0]['l2_rk'], enc_ps[1]['l2_rk'], enc_ps[2]['l2_rk'],
      g1, be1, g2, be2, lw, lb)


def _dec_body(zall_ref, mask_ref, d1w_ref, d1b_ref, d2w_ref, d2b_ref,
              aw_ref, ab_ref, k1_ref, b1_ref, rk1_ref, k2_ref, b2_ref,
              rk2_ref, g1_ref, be1_ref, g2_ref, be2_ref, ow_ref, ob_ref,
              o_ref, xz_ref, hbuf_ref):
    zall = zall_ref[...]
    pooled = jnp.mean(zall, axis=0)
    mr = 1.0 - jnp.mean(jnp.mean(mask_ref[...], axis=2), axis=1,
                        keepdims=True)
    ginp = jnp.concatenate([pooled, mr], axis=-1)
    h = jnp.dot(ginp, d1w_ref[...], precision=PREC_NN) + d1b_ref[...]
    h = 1.0 - jnp.exp(-jnp.square(h))
    h = jnp.dot(h, d2w_ref[...], precision=PREC_NN) + d2b_ref[...]
    h = 1.0 - jnp.exp(-jnp.square(h))
    lg = jnp.dot(h, aw_ref[...], precision=PREC_NN) + ab_ref[...]
    lg = lg - jnp.max(lg, -1, keepdims=True)
    ex = jnp.exp(lg)
    alpha = ex / jnp.sum(ex, -1, keepdims=True)

    zf = (zall[:, :, 0 * L:1 * L] * alpha[:, 0:1]
          + zall[:, :, 1 * L:2 * L] * alpha[:, 1:2]
          + zall[:, :, 2 * L:3 * L] * alpha[:, 2:3])

    xz = jnp.dot(zf.reshape(T * B, L), k1_ref[...], precision=PREC_NN)
    xz_ref[...] = (xz + b1_ref[...]).reshape(T, B, 4 * H)
    _lstm_scan(xz_ref, [rk1_ref[...]], hbuf_ref, B, H)

    h1 = _ln_blocks(hbuf_ref[...], g1_ref[...], be1_ref[...], 1, H)
    xz2 = jnp.dot(h1.reshape(T * B, H), k2_ref[...], precision=PREC_NN)
    xz_ref[...] = (xz2 + b2_ref[...]).reshape(T, B, 4 * H)
    _lstm_scan(xz_ref, [rk2_ref[...]], hbuf_ref, B, H)

    h2 = _ln_blocks(hbuf_ref[...], g2_ref[...], be2_ref[...], 1, H)
    out = jnp.dot(h2.reshape(T * B, H), ow_ref[...], precision=PREC_NN)
    out = jnp.clip(out + ob_ref[...], -5.0, 5.0).reshape(T, B, F)
    for b in range(B):
        o_ref[b] = out[:, b, :]


def _run_decoder(zall, mask, gp, dp):
    return pl.pallas_call(
        _dec_body,
        out_shape=jax.ShapeDtypeStruct((B, T, F), jnp.float32),
        scratch_shapes=[pltpu.VMEM((T, B, 4 * H), jnp.float32),
                        pltpu.VMEM((T, B, H), jnp.float32)],
    )(zall, mask,
      gp['d1_w'], gp['d1_b'][None, :], gp['d2_w'], gp['d2_b'][None, :],
      gp['a_w'], gp['a_b'][None, :],
      dp['l1_k'], dp['l1_b'][None, :], dp['l1_rk'],
      dp['l2_k'], dp['l2_b'][None, :], dp['l2_rk'],
      dp['n1_g'][None, :], dp['n1_b'][None, :],
      dp['n2_g'][None, :], dp['n2_b'][None, :],
      dp['out_w'], dp['out_b'][None, :])


def kernel(x, mask, params):
    zall = _run_encoders(
        x, mask,
        [params['enc_orig'], params['enc_space'], params['enc_time']])
    return _run_decoder(zall, mask, params['gate'], params['dec'])


# speedup vs baseline: 4.8483x; 1.0374x over previous
import functools

import jax
import jax.numpy as jnp
from jax.experimental import pallas as pl
from jax.experimental.pallas import tpu as pltpu

B, T, F = 8, 256, 32
H, L = 128, 32
KNN = 5
PREC = jax.lax.Precision.HIGHEST
PREC_NN = jax.lax.Precision.DEFAULT


def _knn_rows(xp, mp, k):
    n = xp.shape[0]
    u = xp * xp * mp
    v = xp * mp
    cnt = jnp.dot(mp, mp.T, precision=PREC)
    sq = (jnp.dot(u, mp.T, precision=PREC)
          + jnp.dot(mp, u.T, precision=PREC)
          - 2.0 * jnp.dot(v, v.T, precision=PREC))
    ratio = sq / (cnt + 1e-8)
    pos = ratio > 0.0
    safe = jnp.where(pos, ratio, 1.0)
    d = jnp.where(cnt > 0.0, jnp.where(pos, jnp.sqrt(safe), 0.0), jnp.inf)
    col = jax.lax.broadcasted_iota(jnp.int32, (n, n), 1)
    row = jax.lax.broadcasted_iota(jnp.int32, (n, n), 0)
    d = jnp.where(row == col, jnp.inf, d)

    cur = d
    idxs, ds = [], []
    for _ in range(k):
        mval = jnp.min(cur, axis=-1, keepdims=True)
        idx = jnp.min(jnp.where(cur == mval, col, n), -1, keepdims=True)
        idxs.append(idx)
        ds.append(mval)
        cur = jnp.where(col == idx, jnp.inf, cur)
    top_d = jnp.concatenate(ds, axis=-1)
    valid = top_d < jnp.inf
    safe_d = jnp.where(valid, top_d, 0.0)
    vc = jnp.sum(valid.astype(jnp.float32), -1, keepdims=True)
    sigma = jnp.sum(safe_d, -1, keepdims=True) / (vc + 1e-8)
    sigma = jnp.where(vc > 0.0, sigma, 1.0)
    w = jnp.exp(-jnp.square(safe_d) / (jnp.square(sigma) + 1e-8))
    w = jnp.where(valid, w, 0.0)
    w = w / (jnp.sum(w, -1, keepdims=True) + 1e-8)

    wmat = jnp.zeros((n, n), jnp.float32)
    for j in range(k):
        wmat = wmat + jnp.where(col == idxs[j], w[:, j:j + 1], 0.0)
    nv = jnp.dot(wmat, v, precision=PREC)
    ws = jnp.dot(wmat, mp, precision=PREC)
    filled = nv / (ws + 1e-8)
    return jnp.where(mp > 0.0, xp, jnp.where(ws > 0.0, filled, 0.0))


def _cat_rows(vs):
    return jnp.concatenate(vs, axis=0)[None, :]


def _ln_blocks(hseq, g_row, b_row, e_n, units):
    outs = []
    for e in range(e_n):
        sl = hseq[:, :, e * units:(e + 1) * units]
        mu = jnp.mean(sl, -1, keepdims=True)
        var = jnp.mean(jnp.square(sl - mu), -1, keepdims=True)
        outs.append((sl - mu) / jnp.sqrt(var + 1e-3))
    hn = jnp.concatenate(outs, axis=-1) if e_n > 1 else outs[0]
    return hn * g_row + b_row


def _lstm_scan(xz_ref, rks, hbuf_ref, nb, units):
    e_n = len(rks)
    halves = [(rk[:, :2 * units].astype(jnp.bfloat16),
               rk[:, 2 * units:].astype(jnp.bfloat16)) for rk in rks]
    def step(t, carry):
        hs, cs = carry
        xz = xz_ref[pl.ds(t, 1)][0].astype(jnp.float32)
        new_h, new_c = [], []
        for e in range(e_n):
            hb = hs[e].astype(jnp.bfloat16)
            lo, hi = halves[e]
            base = e * 4 * units
            zlo = (xz[:, base:base + 2 * units]
                   + jnp.dot(hb, lo, preferred_element_type=jnp.float32))
            zhi = (xz[:, base + 2 * units:base + 4 * units]
                   + jnp.dot(hb, hi, preferred_element_type=jnp.float32))
            i = jax.nn.sigmoid(zlo[:, :units])
            f = jax.nn.sigmoid(zlo[:, units:])
            g = jnp.tanh(zhi[:, :units])
            o = jax.nn.sigmoid(zhi[:, units:])
            c2 = f * cs[e] + i * g
            new_c.append(c2)
            new_h.append(o * jnp.tanh(c2))
        hcat = new_h[0] if e_n == 1 else jnp.concatenate(new_h, -1)
        hbuf_ref[pl.ds(t, 1)] = hcat[None]
        return (tuple(new_h), tuple(new_c))
    zero = jnp.zeros((nb, units), jnp.float32)
    jax.lax.fori_loop(0, T, step,
                      ((zero,) * e_n, (zero,) * e_n), unroll=4)


def _enc_body(x_ref, m_ref, k1_ref, b1_ref, rk1a_ref, rk1b_ref, rk1c_ref,
              k2_ref, b2_ref, rk2a_ref, rk2b_ref, rk2c_ref,
              g1_ref, be1_ref, g2_ref, be2_ref, lw_ref, lb_ref,
              o_ref, xz_ref, hbuf_ref, xin_ref):
    e_n, eh = 3, 3 * H
    for b in range(B):
        xb = x_ref[b]
        mb = m_ref[b]
        xsp = _knn_rows(xb, mb, KNN)
        xtm = jnp.transpose(
            _knn_rows(jnp.transpose(xb), jnp.transpose(mb), KNN))
        xin_ref[:, b:b + 1, 0 * F:1 * F] = (xb * mb)[:, None, :]
        xin_ref[:, b:b + 1, 1 * F:2 * F] = mb[:, None, :]
        xin_ref[:, b:b + 1, 2 * F:3 * F] = xsp[:, None, :]
        xin_ref[:, b:b + 1, 3 * F:4 * F] = mb[:, None, :]
        xin_ref[:, b:b + 1, 4 * F:5 * F] = xtm[:, None, :]
        xin_ref[:, b:b + 1, 5 * F:6 * F] = mb[:, None, :]
    din = 6 * F
    xin = xin_ref[...]
    xz = jnp.dot(xin.reshape(T * B, din), k1_ref[...], precision=PREC_NN)
    xz_ref[...] = (xz + b1_ref[...]).reshape(T, B, 4 * eh).astype(jnp.bfloat16)
    _lstm_scan(xz_ref, [rk1a_ref[...], rk1b_ref[...], rk1c_ref[...]],
               hbuf_ref, B, H)

    h1 = _ln_blocks(hbuf_ref[...], g1_ref[...], be1_ref[...], e_n, H)
    xz2 = jnp.dot(h1.reshape(T * B, eh), k2_ref[...], precision=PREC_NN)
    xz_ref[...] = (xz2 + b2_ref[...]).reshape(T, B, 4 * eh).astype(jnp.bfloat16)
    _lstm_scan(xz_ref, [rk2a_ref[...], rk2b_ref[...], rk2c_ref[...]],
               hbuf_ref, B, H)

    h2 = _ln_blocks(hbuf_ref[...], g2_ref[...], be2_ref[...], e_n, H)
    z = jnp.dot(h2.reshape(T * B, eh), lw_ref[...], precision=PREC_NN)
    o_ref[...] = (z + lb_ref[...]).reshape(T, B, e_n * L)


def _run_encoders(x, mask, enc_ps):
    eh = 3 * H
    k1 = jax.scipy.linalg.block_diag(*[p['l1_k'] for p in enc_ps])
    b1 = _cat_rows([p['l1_b'] for p in enc_ps])
    k2 = jax.scipy.linalg.block_diag(*[p['l2_k'] for p in enc_ps])
    b2 = _cat_rows([p['l2_b'] for p in enc_ps])
    g1 = _cat_rows([p['n1_g'] for p in enc_ps])
    be1 = _cat_rows([p['n1_b'] for p in enc_ps])
    g2 = _cat_rows([p['n2_g'] for p in enc_ps])
    be2 = _cat_rows([p['n2_b'] for p in enc_ps])
    lw = jax.scipy.linalg.block_diag(*[p['lat_w'] for p in enc_ps])
    lb = _cat_rows([p['lat_b'] for p in enc_ps])
    return pl.pallas_call(
        _enc_body,
        out_shape=jax.ShapeDtypeStruct((T, B, 3 * L), jnp.float32),
        scratch_shapes=[pltpu.VMEM((T, B, 4 * eh), jnp.bfloat16),
                        pltpu.VMEM((T, B, eh), jnp.float32),
                        pltpu.VMEM((T, B, 6 * F), jnp.float32)],
    )(x, mask, k1, b1,
      enc_ps[0]['l1_rk'], enc_ps[1]['l1_rk'], enc_ps[2]['l1_rk'],
      k2, b2,
      enc_ps[0]['l2_rk'], enc_ps[1]['l2_rk'], enc_ps[2]['l2_rk'],
      g1, be1, g2, be2, lw, lb)


def _dec_body(zall_ref, mask_ref, d1w_ref, d1b_ref, d2w_ref, d2b_ref,
              aw_ref, ab_ref, k1_ref, b1_ref, rk1_ref, k2_ref, b2_ref,
              rk2_ref, g1_ref, be1_ref, g2_ref, be2_ref, ow_ref, ob_ref,
              o_ref, xz_ref, hbuf_ref):
    zall = zall_ref[...]
    pooled = jnp.mean(zall, axis=0)
    mr = 1.0 - jnp.mean(jnp.mean(mask_ref[...], axis=2), axis=1,
                        keepdims=True)
    ginp = jnp.concatenate([pooled, mr], axis=-1)
    h = jnp.dot(ginp, d1w_ref[...], precision=PREC_NN) + d1b_ref[...]
    h = 1.0 - jnp.exp(-jnp.square(h))
    h = jnp.dot(h, d2w_ref[...], precision=PREC_NN) + d2b_ref[...]
    h = 1.0 - jnp.exp(-jnp.square(h))
    lg = jnp.dot(h, aw_ref[...], precision=PREC_NN) + ab_ref[...]
    lg = lg - jnp.max(lg, -1, keepdims=True)
    ex = jnp.exp(lg)
    alpha = ex / jnp.sum(ex, -1, keepdims=True)

    zf = (zall[:, :, 0 * L:1 * L] * alpha[:, 0:1]
          + zall[:, :, 1 * L:2 * L] * alpha[:, 1:2]
          + zall[:, :, 2 * L:3 * L] * alpha[:, 2:3])

    xz = jnp.dot(zf.reshape(T * B, L), k1_ref[...], precision=PREC_NN)
    xz_ref[...] = (xz + b1_ref[...]).reshape(T, B, 4 * H).astype(jnp.bfloat16)
    _lstm_scan(xz_ref, [rk1_ref[...]], hbuf_ref, B, H)

    h1 = _ln_blocks(hbuf_ref[...], g1_ref[...], be1_ref[...], 1, H)
    xz2 = jnp.dot(h1.reshape(T * B, H), k2_ref[...], precision=PREC_NN)
    xz_ref[...] = (xz2 + b2_ref[...]).reshape(T, B, 4 * H).astype(jnp.bfloat16)
    _lstm_scan(xz_ref, [rk2_ref[...]], hbuf_ref, B, H)

    h2 = _ln_blocks(hbuf_ref[...], g2_ref[...], be2_ref[...], 1, H)
    out = jnp.dot(h2.reshape(T * B, H), ow_ref[...], precision=PREC_NN)
    out = jnp.clip(out + ob_ref[...], -5.0, 5.0).reshape(T, B, F)
    for b in range(B):
        o_ref[b] = out[:, b, :]


def _run_decoder(zall, mask, gp, dp):
    return pl.pallas_call(
        _dec_body,
        out_shape=jax.ShapeDtypeStruct((B, T, F), jnp.float32),
        scratch_shapes=[pltpu.VMEM((T, B, 4 * H), jnp.bfloat16),
                        pltpu.VMEM((T, B, H), jnp.float32)],
    )(zall, mask,
      gp['d1_w'], gp['d1_b'][None, :], gp['d2_w'], gp['d2_b'][None, :],
      gp['a_w'], gp['a_b'][None, :],
      dp['l1_k'], dp['l1_b'][None, :], dp['l1_rk'],
      dp['l2_k'], dp['l2_b'][None, :], dp['l2_rk'],
      dp['n1_g'][None, :], dp['n1_b'][None, :],
      dp['n2_g'][None, :], dp['n2_b'][None, :],
      dp['out_w'], dp['out_b'][None, :])


def kernel(x, mask, params):
    zall = _run_encoders(
        x, mask,
        [params['enc_orig'], params['enc_space'], params['enc_time']])
    return _run_decoder(zall, mask, params['gate'], params['dec'])


# speedup vs baseline: 5.0017x; 1.0316x over previous
import functools

import jax
import jax.numpy as jnp
from jax.experimental import pallas as pl
from jax.experimental.pallas import tpu as pltpu

B, T, F = 8, 256, 32
H, L = 128, 32
KNN = 5
PREC = jax.lax.Precision.HIGHEST
PREC_NN = jax.lax.Precision.DEFAULT


def _knn_rows(xp, mp, k):
    n = xp.shape[0]
    u = xp * xp * mp
    v = xp * mp
    cnt = jnp.dot(mp, mp.T, precision=PREC)
    sq = (jnp.dot(u, mp.T, precision=PREC)
          + jnp.dot(mp, u.T, precision=PREC)
          - 2.0 * jnp.dot(v, v.T, precision=PREC))
    ratio = sq / (cnt + 1e-8)
    pos = ratio > 0.0
    safe = jnp.where(pos, ratio, 1.0)
    d = jnp.where(cnt > 0.0, jnp.where(pos, jnp.sqrt(safe), 0.0), jnp.inf)
    col = jax.lax.broadcasted_iota(jnp.int32, (n, n), 1)
    row = jax.lax.broadcasted_iota(jnp.int32, (n, n), 0)
    d = jnp.where(row == col, jnp.inf, d)

    cur = d
    ds = []
    for _ in range(k):
        mval = jnp.min(cur, axis=-1, keepdims=True)
        ds.append(mval)
        cur = jnp.where(cur == mval, jnp.inf, cur)
    top_d = jnp.concatenate(ds, axis=-1)
    t5 = ds[-1]
    valid = top_d < jnp.inf
    safe_d = jnp.where(valid, top_d, 0.0)
    vc = jnp.sum(valid.astype(jnp.float32), -1, keepdims=True)
    sigma = jnp.sum(safe_d, -1, keepdims=True) / (vc + 1e-8)
    sigma = jnp.where(vc > 0.0, sigma, 1.0)
    sig2 = jnp.square(sigma) + 1e-8
    w = jnp.exp(-jnp.square(safe_d) / sig2)
    w = jnp.where(valid, w, 0.0)
    zsum = jnp.sum(w, -1, keepdims=True) + 1e-8

    wmat = jnp.where(d <= t5, jnp.exp(-jnp.square(d) / sig2), 0.0) / zsum
    nv = jnp.dot(wmat, v, precision=PREC)
    ws = jnp.dot(wmat, mp, precision=PREC)
    filled = nv / (ws + 1e-8)
    return jnp.where(mp > 0.0, xp, jnp.where(ws > 0.0, filled, 0.0))


def _cat_rows(vs):
    return jnp.concatenate(vs, axis=0)[None, :]


def _ln_blocks(hseq, g_row, b_row, e_n, units):
    outs = []
    for e in range(e_n):
        sl = hseq[:, :, e * units:(e + 1) * units]
        mu = jnp.mean(sl, -1, keepdims=True)
        var = jnp.mean(jnp.square(sl - mu), -1, keepdims=True)
        outs.append((sl - mu) / jnp.sqrt(var + 1e-3))
    hn = jnp.concatenate(outs, axis=-1) if e_n > 1 else outs[0]
    return hn * g_row + b_row


def _lstm_scan(xz_ref, rks, hbuf_ref, nb, units):
    e_n = len(rks)
    halves = [(rk[:, :2 * units].astype(jnp.bfloat16),
               rk[:, 2 * units:].astype(jnp.bfloat16)) for rk in rks]
    def step(t, carry):
        hs, cs = carry
        xz = xz_ref[pl.ds(t, 1)][0].astype(jnp.float32)
        new_h, new_c = [], []
        for e in range(e_n):
            hb = hs[e].astype(jnp.bfloat16)
            lo, hi = halves[e]
            base = e * 4 * units
            zlo = (xz[:, base:base + 2 * units]
                   + jnp.dot(hb, lo, preferred_element_type=jnp.float32))
            zhi = (xz[:, base + 2 * units:base + 4 * units]
                   + jnp.dot(hb, hi, preferred_element_type=jnp.float32))
            i = jax.nn.sigmoid(zlo[:, :units])
            f = jax.nn.sigmoid(zlo[:, units:])
            g = jnp.tanh(zhi[:, :units])
            o = jax.nn.sigmoid(zhi[:, units:])
            c2 = f * cs[e] + i * g
            new_c.append(c2)
            new_h.append(o * jnp.tanh(c2))
        hcat = new_h[0] if e_n == 1 else jnp.concatenate(new_h, -1)
        hbuf_ref[pl.ds(t, 1)] = hcat[None]
        return (tuple(new_h), tuple(new_c))
    zero = jnp.zeros((nb, units), jnp.float32)
    jax.lax.fori_loop(0, T, step,
                      ((zero,) * e_n, (zero,) * e_n), unroll=4)


def _enc_body(x_ref, m_ref, k1_ref, b1_ref, rk1a_ref, rk1b_ref, rk1c_ref,
              k2_ref, b2_ref, rk2a_ref, rk2b_ref, rk2c_ref,
              g1_ref, be1_ref, g2_ref, be2_ref, lw_ref, lb_ref,
              o_ref, xz_ref, hbuf_ref, xin_ref):
    e_n, eh = 3, 3 * H
    for b in range(B):
        xb = x_ref[b]
        mb = m_ref[b]
        xsp = _knn_rows(xb, mb, KNN)
        xtm = jnp.transpose(
            _knn_rows(jnp.transpose(xb), jnp.transpose(mb), KNN))
        xin_ref[:, b:b + 1, 0 * F:1 * F] = (xb * mb)[:, None, :]
        xin_ref[:, b:b + 1, 1 * F:2 * F] = mb[:, None, :]
        xin_ref[:, b:b + 1, 2 * F:3 * F] = xsp[:, None, :]
        xin_ref[:, b:b + 1, 3 * F:4 * F] = mb[:, None, :]
        xin_ref[:, b:b + 1, 4 * F:5 * F] = xtm[:, None, :]
        xin_ref[:, b:b + 1, 5 * F:6 * F] = mb[:, None, :]
    din = 6 * F
    xin = xin_ref[...]
    xz = jnp.dot(xin.reshape(T * B, din), k1_ref[...], precision=PREC_NN)
    xz_ref[...] = (xz + b1_ref[...]).reshape(T, B, 4 * eh).astype(jnp.bfloat16)
    _lstm_scan(xz_ref, [rk1a_ref[...], rk1b_ref[...], rk1c_ref[...]],
               hbuf_ref, B, H)

    h1 = _ln_blocks(hbuf_ref[...], g1_ref[...], be1_ref[...], e_n, H)
    xz2 = jnp.dot(h1.reshape(T * B, eh), k2_ref[...], precision=PREC_NN)
    xz_ref[...] = (xz2 + b2_ref[...]).reshape(T, B, 4 * eh).astype(jnp.bfloat16)
    _lstm_scan(xz_ref, [rk2a_ref[...], rk2b_ref[...], rk2c_ref[...]],
               hbuf_ref, B, H)

    h2 = _ln_blocks(hbuf_ref[...], g2_ref[...], be2_ref[...], e_n, H)
    z = jnp.dot(h2.reshape(T * B, eh), lw_ref[...], precision=PREC_NN)
    o_ref[...] = (z + lb_ref[...]).reshape(T, B, e_n * L)


def _run_encoders(x, mask, enc_ps):
    eh = 3 * H
    k1 = jax.scipy.linalg.block_diag(*[p['l1_k'] for p in enc_ps])
    b1 = _cat_rows([p['l1_b'] for p in enc_ps])
    k2 = jax.scipy.linalg.block_diag(*[p['l2_k'] for p in enc_ps])
    b2 = _cat_rows([p['l2_b'] for p in enc_ps])
    g1 = _cat_rows([p['n1_g'] for p in enc_ps])
    be1 = _cat_rows([p['n1_b'] for p in enc_ps])
    g2 = _cat_rows([p['n2_g'] for p in enc_ps])
    be2 = _cat_rows([p['n2_b'] for p in enc_ps])
    lw = jax.scipy.linalg.block_diag(*[p['lat_w'] for p in enc_ps])
    lb = _cat_rows([p['lat_b'] for p in enc_ps])
    return pl.pallas_call(
        _enc_body,
        out_shape=jax.ShapeDtypeStruct((T, B, 3 * L), jnp.float32),
        scratch_shapes=[pltpu.VMEM((T, B, 4 * eh), jnp.bfloat16),
                        pltpu.VMEM((T, B, eh), jnp.float32),
                        pltpu.VMEM((T, B, 6 * F), jnp.float32)],
    )(x, mask, k1, b1,
      enc_ps[0]['l1_rk'], enc_ps[1]['l1_rk'], enc_ps[2]['l1_rk'],
      k2, b2,
      enc_ps[0]['l2_rk'], enc_ps[1]['l2_rk'], enc_ps[2]['l2_rk'],
      g1, be1, g2, be2, lw, lb)


def _dec_body(zall_ref, mask_ref, d1w_ref, d1b_ref, d2w_ref, d2b_ref,
              aw_ref, ab_ref, k1_ref, b1_ref, rk1_ref, k2_ref, b2_ref,
              rk2_ref, g1_ref, be1_ref, g2_ref, be2_ref, ow_ref, ob_ref,
              o_ref, xz_ref, hbuf_ref):
    zall = zall_ref[...]
    pooled = jnp.mean(zall, axis=0)
    mr = 1.0 - jnp.mean(jnp.mean(mask_ref[...], axis=2), axis=1,
                        keepdims=True)
    ginp = jnp.concatenate([pooled, mr], axis=-1)
    h = jnp.dot(ginp, d1w_ref[...], precision=PREC_NN) + d1b_ref[...]
    h = 1.0 - jnp.exp(-jnp.square(h))
    h = jnp.dot(h, d2w_ref[...], precision=PREC_NN) + d2b_ref[...]
    h = 1.0 - jnp.exp(-jnp.square(h))
    lg = jnp.dot(h, aw_ref[...], precision=PREC_NN) + ab_ref[...]
    lg = lg - jnp.max(lg, -1, keepdims=True)
    ex = jnp.exp(lg)
    alpha = ex / jnp.sum(ex, -1, keepdims=True)

    zf = (zall[:, :, 0 * L:1 * L] * alpha[:, 0:1]
          + zall[:, :, 1 * L:2 * L] * alpha[:, 1:2]
          + zall[:, :, 2 * L:3 * L] * alpha[:, 2:3])

    xz = jnp.dot(zf.reshape(T * B, L), k1_ref[...], precision=PREC_NN)
    xz_ref[...] = (xz + b1_ref[...]).reshape(T, B, 4 * H).astype(jnp.bfloat16)
    _lstm_scan(xz_ref, [rk1_ref[...]], hbuf_ref, B, H)

    h1 = _ln_blocks(hbuf_ref[...], g1_ref[...], be1_ref[...], 1, H)
    xz2 = jnp.dot(h1.reshape(T * B, H), k2_ref[...], precision=PREC_NN)
    xz_ref[...] = (xz2 + b2_ref[...]).reshape(T, B, 4 * H).astype(jnp.bfloat16)
    _lstm_scan(xz_ref, [rk2_ref[...]], hbuf_ref, B, H)

    h2 = _ln_blocks(hbuf_ref[...], g2_ref[...], be2_ref[...], 1, H)
    out = jnp.dot(h2.reshape(T * B, H), ow_ref[...], precision=PREC_NN)
    out = jnp.clip(out + ob_ref[...], -5.0, 5.0).reshape(T, B, F)
    for b in range(B):
        o_ref[b] = out[:, b, :]


def _run_decoder(zall, mask, gp, dp):
    return pl.pallas_call(
        _dec_body,
        out_shape=jax.ShapeDtypeStruct((B, T, F), jnp.float32),
        scratch_shapes=[pltpu.VMEM((T, B, 4 * H), jnp.bfloat16),
                        pltpu.VMEM((T, B, H), jnp.float32)],
    )(zall, mask,
      gp['d1_w'], gp['d1_b'][None, :], gp['d2_w'], gp['d2_b'][None, :],
      gp['a_w'], gp['a_b'][None, :],
      dp['l1_k'], dp['l1_b'][None, :], dp['l1_rk'],
      dp['l2_k'], dp['l2_b'][None, :], dp['l2_rk'],
      dp['n1_g'][None, :], dp['n1_b'][None, :],
      dp['n2_g'][None, :], dp['n2_b'][None, :],
      dp['out_w'], dp['out_b'][None, :])


def kernel(x, mask, params):
    zall = _run_encoders(
        x, mask,
        [params['enc_orig'], params['enc_space'], params['enc_time']])
    return _run_decoder(zall, mask, params['gate'], params['dec'])
